# baseline (speedup 1.0000x reference)
import jax
import jax.numpy as jnp
from jax import lax
from jax.experimental import pallas as pl
from jax.experimental.pallas import tpu as pltpu
from jax.experimental.pallas import tpu_sc as plsc

NUM_ENTITIES = 100000
NUM_RELATIONS = 1000
EMBED_DIM = 64
BATCH = 16384

NC = 2
NS = 16
L = 16
NW = NC * NS
B_PER_W = BATCH // NW
CHUNK = 256
N_CHUNKS = B_PER_W // CHUNK
CHUNKS = EMBED_DIM // L
W = 2 * EMBED_DIM


def _vsqrt(x):
    i = plsc.bitcast(x, jnp.int32)
    y = plsc.bitcast(jnp.int32(0x5F3759DF) - (i >> 1), jnp.float32)
    for _ in range(3):
        y = y * (1.5 - 0.5 * x * y * y)
    return x * y


def _body(heads_hbm, relations_hbm, tails_hbm, ent_hbm, rel_hbm, out_hbm,
          idx_h, idx_r, idx_t, pidx_h, pidx_r, pidx_t,
          h_rows, r_rows, t_rows, out_v, sem):
    wid = lax.axis_index("s") * NC + lax.axis_index("c")
    base = wid * B_PER_W

    pltpu.sync_copy(heads_hbm.at[pl.ds(base, B_PER_W)], idx_h)
    pltpu.sync_copy(relations_hbm.at[pl.ds(base, B_PER_W)], idx_r)
    pltpu.sync_copy(tails_hbm.at[pl.ds(base, B_PER_W)], idx_t)

    def halve(k, _):
        s = pl.ds(k * L, L)
        pidx_h[s] = idx_h[s] >> 1
        pidx_r[s] = idx_r[s] >> 1
        pidx_t[s] = idx_t[s] >> 1
        return 0

    lax.fori_loop(0, B_PER_W // L, halve, 0)

    iota = lax.iota(jnp.int32, L)

    def chunk(c, _):
        cb = c * CHUNK
        c1 = pltpu.async_copy(ent_hbm.at[pidx_h.at[pl.ds(cb, CHUNK)]],
                              h_rows, sem)
        c2 = pltpu.async_copy(rel_hbm.at[pidx_r.at[pl.ds(cb, CHUNK)]],
                              r_rows, sem)
        c3 = pltpu.async_copy(ent_hbm.at[pidx_t.at[pl.ds(cb, CHUNK)]],
                              t_rows, sem)
        c1.wait()
        c2.wait()
        c3.wait()

        def group(g, _):
            gb = g * L
            s = pl.ds(cb + gb, L)
            voh = (idx_h[s] & 1) * EMBED_DIM
            vor = (idx_r[s] & 1) * EMBED_DIM
            vot = (idx_t[s] & 1) * EMBED_DIM
            gv = jnp.zeros((L,), jnp.float32)
            for j in range(L):
                i = gb + j
                oh = voh[j]
                orr = vor[j]
                ot = vot[j]
                acc = jnp.zeros((L,), jnp.float32)
                for k in range(CHUNKS):
                    h = h_rows[i, pl.ds(oh + k * L, L)]
                    r = r_rows[i, pl.ds(orr + k * L, L)]
                    t = t_rows[i, pl.ds(ot + k * L, L)]
                    df = h + r - t + 1e-6
                    acc = acc + df * df
                gv = jnp.where(iota == j, jnp.sum(acc), gv)
            out_v[pl.ds(cb + gb, L)] = _vsqrt(gv)
            return 0

        lax.fori_loop(0, CHUNK // L, group, 0)
        return 0

    lax.fori_loop(0, N_CHUNKS, chunk, 0)

    pltpu.sync_copy(out_v, out_hbm.at[pl.ds(base, B_PER_W)])


@jax.jit
def _transe(heads, relations, tails, entity_emb, relation_emb):
    ent2 = entity_emb.reshape(NUM_ENTITIES // 2, W)
    rel2 = relation_emb.reshape(NUM_RELATIONS // 2, W)
    mesh = plsc.VectorSubcoreMesh(
        core_axis_name="c", subcore_axis_name="s", num_cores=NC,
        num_subcores=NS)
    return pl.kernel(
        _body,
        out_type=jax.ShapeDtypeStruct((BATCH,), jnp.float32),
        mesh=mesh,
        scratch_types=[
            pltpu.VMEM((B_PER_W,), jnp.int32),
            pltpu.VMEM((B_PER_W,), jnp.int32),
            pltpu.VMEM((B_PER_W,), jnp.int32),
            pltpu.VMEM((B_PER_W,), jnp.int32),
            pltpu.VMEM((B_PER_W,), jnp.int32),
            pltpu.VMEM((B_PER_W,), jnp.int32),
            pltpu.VMEM((CHUNK, W), jnp.float32),
            pltpu.VMEM((CHUNK, W), jnp.float32),
            pltpu.VMEM((CHUNK, W), jnp.float32),
            pltpu.VMEM((B_PER_W,), jnp.float32),
            pltpu.SemaphoreType.DMA,
        ],
        compiler_params=pltpu.CompilerParams(
            needs_layout_passes=False, use_tc_tiling_on_sc=True),
    )(heads, relations, tails, ent2, rel2)


def kernel(heads, relations, tails, entity_emb, relation_emb):
    return _transe(heads, relations, tails, entity_emb, relation_emb)


# speedup vs baseline: 1.3716x; 1.3716x over previous
import jax
import jax.numpy as jnp
from jax import lax
from jax.experimental import pallas as pl
from jax.experimental.pallas import tpu as pltpu
from jax.experimental.pallas import tpu_sc as plsc

NUM_ENTITIES = 100000
NUM_RELATIONS = 1000
EMBED_DIM = 64
BATCH = 16384

NC = 2
NS = 16
L = 16
NW = NC * NS
B_PER_W = BATCH // NW
N_GROUPS = B_PER_W // L
CHUNKS = EMBED_DIM // L


def _vsqrt(x):
    i = plsc.bitcast(x, jnp.int32)
    y = plsc.bitcast(jnp.int32(0x5F3759DF) - (i >> 1), jnp.float32)
    for _ in range(3):
        y = y * (1.5 - 0.5 * x * y * y)
    return x * y


def _body(heads_hbm, relations_hbm, tails_hbm, ent_hbm, rel_hbm, out_hbm,
          idx_h, idx_r, idx_t, h_rows, r_rows, t_rows, out_v,
          sem_h, sem_r, sem_t):
    wid = lax.axis_index("s") * NC + lax.axis_index("c")
    base = wid * B_PER_W

    pltpu.sync_copy(heads_hbm.at[pl.ds(base, B_PER_W)], idx_h)
    pltpu.sync_copy(relations_hbm.at[pl.ds(base, B_PER_W)], idx_r)
    pltpu.sync_copy(tails_hbm.at[pl.ds(base, B_PER_W)], idx_t)

    def fire(g, slot):
        gb = g * L
        s = pl.ds(gb, L)
        vh = idx_h[s]
        vr = idx_r[s]
        vt = idx_t[s]
        for j in range(L):
            pltpu.async_copy(ent_hbm.at[vh[j]], h_rows.at[slot, j], sem_h)
            pltpu.async_copy(rel_hbm.at[vr[j]], r_rows.at[slot, j], sem_r)
            pltpu.async_copy(ent_hbm.at[vt[j]], t_rows.at[slot, j], sem_t)

    def drain():
        pltpu.make_async_copy(
            ent_hbm.at[pl.ds(0, L)], h_rows.at[0], sem_h).wait()
        pltpu.make_async_copy(
            rel_hbm.at[pl.ds(0, L)], r_rows.at[0], sem_r).wait()
        pltpu.make_async_copy(
            ent_hbm.at[pl.ds(0, L)], t_rows.at[0], sem_t).wait()

    iota = lax.iota(jnp.int32, L)

    def compute(g, slot):
        gv = jnp.zeros((L,), jnp.float32)
        for j in range(L):
            acc = jnp.zeros((L,), jnp.float32)
            for k in range(CHUNKS):
                h = h_rows[slot, j, pl.ds(k * L, L)]
                r = r_rows[slot, j, pl.ds(k * L, L)]
                t = t_rows[slot, j, pl.ds(k * L, L)]
                df = h + r - t + 1e-6
                acc = acc + df * df
            gv = jnp.where(iota == j, jnp.sum(acc), gv)
        out_v[pl.ds(g * L, L)] = _vsqrt(gv)

    fire(0, 0)

    def step(g, _):
        fire(g + 1, (g + 1) & 1)
        drain()
        compute(g, g & 1)
        return 0

    lax.fori_loop(0, N_GROUPS - 1, step, 0)
    drain()
    compute(N_GROUPS - 1, (N_GROUPS - 1) & 1)

    pltpu.sync_copy(out_v, out_hbm.at[pl.ds(base, B_PER_W)])


@jax.jit
def _transe(heads, relations, tails, entity_emb, relation_emb):
    mesh = plsc.VectorSubcoreMesh(
        core_axis_name="c", subcore_axis_name="s", num_cores=NC,
        num_subcores=NS)
    return pl.kernel(
        _body,
        out_type=jax.ShapeDtypeStruct((BATCH,), jnp.float32),
        mesh=mesh,
        scratch_types=[
            pltpu.VMEM((B_PER_W,), jnp.int32),
            pltpu.VMEM((B_PER_W,), jnp.int32),
            pltpu.VMEM((B_PER_W,), jnp.int32),
            pltpu.VMEM((2, L, EMBED_DIM), jnp.float32),
            pltpu.VMEM((2, L, EMBED_DIM), jnp.float32),
            pltpu.VMEM((2, L, EMBED_DIM), jnp.float32),
            pltpu.VMEM((B_PER_W,), jnp.float32),
            pltpu.SemaphoreType.DMA,
            pltpu.SemaphoreType.DMA,
            pltpu.SemaphoreType.DMA,
        ],
        compiler_params=pltpu.CompilerParams(
            needs_layout_passes=False, use_tc_tiling_on_sc=True),
    )(heads, relations, tails, entity_emb, relation_emb)


def kernel(heads, relations, tails, entity_emb, relation_emb):
    return _transe(heads, relations, tails, entity_emb, relation_emb)
